# Initial kernel scaffold; baseline (speedup 1.0000x reference)
import jax
import jax.numpy as jnp
from jax.experimental import pallas as pl


def kernel(x, adj, W1, b1, W2, b2, W3, b3, W4, b4, W5, b5, W6, b6):
    raise NotImplementedError("write your pallas kernel here")


# speedup vs baseline: 1.3043x; 1.3043x over previous
import jax
import jax.numpy as jnp
from jax.experimental import pallas as pl
from jax.experimental.pallas import tpu as pltpu

F = 128
BR1 = 320
BR = 640
_PAD = 640


def _rows_lt(n, base_rows, shape):
    rows = base_rows + jax.lax.broadcasted_iota(jnp.int32, shape, 0)
    return rows < n


def _support0_body(n, x_ref, w_ref, o_ref):
    r = pl.program_id(0)
    s = jnp.dot(x_ref[...].astype(jnp.bfloat16), w_ref[...],
                preferred_element_type=jnp.float32)
    br = o_ref.shape[0]
    o_ref[...] = jnp.where(_rows_lt(n, r * br, s.shape), s, 0.0
                           ).astype(jnp.bfloat16)


def _layer1_body(n, adj_ref, s_ref, w_ref, b_ref, aq_ref, o_ref):
    r = pl.program_id(0)
    a = adj_ref[...]
    rows = _rows_lt(n, r * BR1, a.shape)
    cols = jax.lax.broadcasted_iota(jnp.int32, a.shape, 1) < n
    aq = jnp.where(rows & cols, a, 0.0).astype(jnp.bfloat16)
    aq_ref[...] = aq
    acc = jnp.dot(aq, s_ref[...], preferred_element_type=jnp.float32)
    h = jnp.maximum(acc + b_ref[...], 0.0)
    s2 = jnp.dot(h.astype(jnp.bfloat16), w_ref[...],
                 preferred_element_type=jnp.float32)
    o_ref[...] = jnp.where(_rows_lt(n, r * BR1, s2.shape), s2, 0.0
                           ).astype(jnp.bfloat16)


def _mid_body(n, aq_ref, s_ref, w_ref, b_ref, o_ref):
    r = pl.program_id(0)
    acc = jnp.dot(aq_ref[...], s_ref[...], preferred_element_type=jnp.float32)
    h = jnp.maximum(acc + b_ref[...], 0.0)
    s2 = jnp.dot(h.astype(jnp.bfloat16), w_ref[...],
                 preferred_element_type=jnp.float32)
    o_ref[...] = jnp.where(_rows_lt(n, r * BR, s2.shape), s2, 0.0
                           ).astype(jnp.bfloat16)


def _last_body(aq_ref, s_ref, b_ref, o_ref):
    acc = jnp.dot(aq_ref[...], s_ref[...], preferred_element_type=jnp.float32)
    o_ref[...] = acc + b_ref[...]


def _cparams():
    return pltpu.CompilerParams(dimension_semantics=("arbitrary",))


def kernel(x, adj, W1, b1, W2, b2, W3, b3, W4, b4, W5, b5, W6, b6):
    n = x.shape[0]
    np_ = ((n + _PAD - 1) // _PAD) * _PAD
    f32 = jnp.float32
    bf16 = jnp.bfloat16
    wb = [w.astype(bf16) for w in (W1, W2, W3, W4, W5, W6)]
    bs = [b.reshape(1, F) for b in (b1, b2, b3, b4, b5, b6)]

    full_s = pl.BlockSpec((np_, F), lambda r: (0, 0))
    full_w = pl.BlockSpec((F, F), lambda r: (0, 0))
    full_b = pl.BlockSpec((1, F), lambda r: (0, 0))

    import functools
    s = pl.pallas_call(
        functools.partial(_support0_body, n),
        grid=(np_ // BR,),
        in_specs=[pl.BlockSpec((BR, F), lambda r: (r, 0)), full_w],
        out_specs=pl.BlockSpec((BR, F), lambda r: (r, 0)),
        out_shape=jax.ShapeDtypeStruct((np_, F), bf16),
        compiler_params=_cparams(),
    )(x, wb[0])

    adjq, s = pl.pallas_call(
        functools.partial(_layer1_body, n),
        grid=(np_ // BR1,),
        in_specs=[pl.BlockSpec((BR1, np_), lambda r: (r, 0)),
                  full_s, full_w, full_b],
        out_specs=[pl.BlockSpec((BR1, np_), lambda r: (r, 0)),
                   pl.BlockSpec((BR1, F), lambda r: (r, 0))],
        out_shape=[jax.ShapeDtypeStruct((np_, np_), bf16),
                   jax.ShapeDtypeStruct((np_, F), bf16)],
        compiler_params=_cparams(),
    )(adj, s, wb[1], bs[0])

    for li in (1, 2, 3, 4):
        s = pl.pallas_call(
            functools.partial(_mid_body, n),
            grid=(np_ // BR,),
            in_specs=[pl.BlockSpec((BR, np_), lambda r: (r, 0)),
                      full_s, full_w, full_b],
            out_specs=pl.BlockSpec((BR, F), lambda r: (r, 0)),
            out_shape=jax.ShapeDtypeStruct((np_, F), bf16),
            compiler_params=_cparams(),
        )(adjq, s, wb[li + 1], bs[li])

    out = pl.pallas_call(
        _last_body,
        grid=(np_ // BR,),
        in_specs=[pl.BlockSpec((BR, np_), lambda r: (r, 0)),
                  full_s, full_b],
        out_specs=pl.BlockSpec((BR, F), lambda r: (r, 0)),
        out_shape=jax.ShapeDtypeStruct((np_, F), f32),
        compiler_params=_cparams(),
    )(adjq, s, bs[5])

    return out[:n]
